# Initial kernel scaffold; baseline (speedup 1.0000x reference)
import jax
import jax.numpy as jnp
from jax import lax
from jax.experimental import pallas as pl
from jax.experimental.pallas import tpu as pltpu
from jax.experimental.pallas import tpu_sc as plsc

NC = 2
NS = 16
NW = NC * NS
LANE = 16
BATCH = 128


def _ceil_to(a, m):
    return -(-a // m) * m


def _build_deg_kernel(n_pad, nb, nb_tot):
    rows_t = n_pad // NS
    mesh = plsc.VectorSubcoreMesh(core_axis_name="c", subcore_axis_name="s")

    def body(dst_hbm, out_hbm, dst_v, ones_v, stage_v, deg_sh):
        c = lax.axis_index("c")
        s = lax.axis_index("s")
        wid = c * NS + s

        def fill_ones(i, carry):
            ones_v[pl.ds(i * LANE, LANE)] = jnp.ones((LANE,), jnp.float32)
            return carry

        lax.fori_loop(0, BATCH // LANE, fill_ones, 0)

        def fill_zero(i, carry):
            stage_v[pl.ds(i * LANE, LANE)] = jnp.zeros((LANE,), jnp.float32)
            return carry

        lax.fori_loop(0, rows_t // LANE, fill_zero, 0)

        pltpu.sync_copy(stage_v, deg_sh.at[pl.ds(s * rows_t, rows_t)])
        pltpu.sync_copy(dst_hbm.at[pl.ds(wid * nb_tot, nb)], dst_v)
        plsc.subcore_barrier()

        def scat(b, carry):
            pltpu.sync_copy(ones_v, deg_sh.at[dst_v.at[b]], add=True)
            return carry

        lax.fori_loop(0, nb, scat, 0)
        plsc.subcore_barrier()

        pltpu.sync_copy(deg_sh.at[pl.ds(s * rows_t, rows_t)], stage_v)
        pltpu.sync_copy(stage_v, out_hbm.at[pl.ds(wid * rows_t, rows_t)])

    return pl.kernel(
        body,
        out_type=jax.ShapeDtypeStruct((NC * n_pad,), jnp.float32),
        mesh=mesh,
        scratch_types=[
            pltpu.VMEM((nb, BATCH), jnp.int32),
            pltpu.VMEM((BATCH,), jnp.float32),
            pltpu.VMEM((rows_t,), jnp.float32),
            pltpu.VMEM_SHARED((n_pad,), jnp.float32),
        ],
    )


def _build_agg_kernel(n_pad, nb, nb_tot, d):
    rows_t = n_pad // NS
    n_cp = rows_t // BATCH
    mesh = plsc.VectorSubcoreMesh(core_axis_name="c", subcore_axis_name="s")

    def body(srcix_hbm, dstix_hbm, xs_hbm, out_hbm,
             src_v, dst_v, buf0, buf1, agg_sh, sem0, sem1):
        c = lax.axis_index("c")
        s = lax.axis_index("s")
        wid = c * NS + s
        dl = d // LANE

        def z(i, carry):
            buf0[i // dl, pl.ds((i % dl) * LANE, LANE)] = (
                jnp.zeros((LANE,), jnp.float32))
            return carry

        lax.fori_loop(0, BATCH * dl, z, 0)

        def zc(i, carry):
            pltpu.sync_copy(buf0, agg_sh.at[pl.ds(s * rows_t + i * BATCH, BATCH)])
            return carry

        lax.fori_loop(0, n_cp, zc, 0)

        pltpu.sync_copy(srcix_hbm.at[pl.ds(wid * nb_tot, nb_tot)], src_v)
        pltpu.sync_copy(dstix_hbm.at[pl.ds(wid * nb_tot, nb)], dst_v)
        plsc.subcore_barrier()

        pltpu.async_copy(xs_hbm.at[src_v.at[0]], buf0, sem0)
        pltpu.async_copy(xs_hbm.at[src_v.at[1]], buf1, sem1)

        def step(g, carry):
            b0 = 2 * g
            pltpu.make_async_copy(xs_hbm.at[src_v.at[b0]], buf0, sem0).wait()
            pltpu.sync_copy(buf0, agg_sh.at[dst_v.at[b0]], add=True)
            pltpu.async_copy(xs_hbm.at[src_v.at[b0 + 2]], buf0, sem0)
            b1 = b0 + 1
            pltpu.make_async_copy(xs_hbm.at[src_v.at[b1]], buf1, sem1).wait()
            pltpu.sync_copy(buf1, agg_sh.at[dst_v.at[b1]], add=True)
            pltpu.async_copy(xs_hbm.at[src_v.at[b1 + 2]], buf1, sem1)
            return carry

        lax.fori_loop(0, nb // 2, step, 0)
        pltpu.make_async_copy(xs_hbm.at[src_v.at[0]], buf0, sem0).wait()
        pltpu.make_async_copy(xs_hbm.at[src_v.at[1]], buf1, sem1).wait()
        plsc.subcore_barrier()

        def cp(i, carry):
            pltpu.sync_copy(agg_sh.at[pl.ds(s * rows_t + i * BATCH, BATCH)], buf0)
            pltpu.sync_copy(buf0, out_hbm.at[pl.ds(wid * rows_t + i * BATCH, BATCH)])
            return carry

        lax.fori_loop(0, n_cp, cp, 0)

    return pl.kernel(
        body,
        out_type=jax.ShapeDtypeStruct((NC * n_pad, d), jnp.float32),
        mesh=mesh,
        scratch_types=[
            pltpu.VMEM((nb_tot, BATCH), jnp.int32),
            pltpu.VMEM((nb, BATCH), jnp.int32),
            pltpu.VMEM((BATCH, d), jnp.float32),
            pltpu.VMEM((BATCH, d), jnp.float32),
            pltpu.VMEM_SHARED((n_pad, d), jnp.float32),
            pltpu.SemaphoreType.DMA,
            pltpu.SemaphoreType.DMA,
        ],
    )


def _norm_body(degt_ref, x_ref, dinv_ref, xs_ref):
    deg = degt_ref[:, 0:1] + degt_ref[:, 1:2] + 1.0
    dinv = lax.rsqrt(deg)
    dinv_ref[...] = dinv
    xs_ref[...] = x_ref[...] * dinv


def _out_body(p_ref, xs_ref, dinv_ref, w_ref, b_ref, o_ref):
    m = p_ref[0] + p_ref[1] + xs_ref[...]
    y = jnp.dot(m, w_ref[...], preferred_element_type=jnp.float32)
    o_ref[...] = jnp.maximum(y * dinv_ref[...] + b_ref[...], 0.0)


def kernel(x, edge_index, W1, b1):
    n, d_in = x.shape
    d_hid = W1.shape[1]
    e = edge_index.shape[1]

    n_pad = _ceil_to(n + 1, NS * BATCH)
    pad_rows = n_pad - n
    nb = -(-e // (NW * BATCH))
    nb += nb % 2
    e_pad = NW * nb * BATCH
    nb_tot = nb + 2

    src = edge_index[0]
    dst = edge_index[1]
    pad_idx = n + (jnp.arange(e_pad - e, dtype=jnp.int32) % pad_rows)
    src_p = jnp.concatenate([src, pad_idx]).reshape(NW, nb, BATCH)
    dst_p = jnp.concatenate([dst, pad_idx]).reshape(NW, nb, BATCH)
    dummy = n + (jnp.arange(2 * BATCH, dtype=jnp.int32) % pad_rows)
    dummy = jnp.broadcast_to(dummy.reshape(1, 2, BATCH), (NW, 2, BATCH))
    src_full = jnp.concatenate([src_p, dummy], axis=1).reshape(NW * nb_tot, BATCH)
    dst_full = jnp.concatenate([dst_p, dummy], axis=1).reshape(NW * nb_tot, BATCH)
    x_pad = jnp.pad(x, ((0, pad_rows), (0, 0)))

    degp = _build_deg_kernel(n_pad, nb, nb_tot)(dst_full)
    degt = degp.reshape(NC, n_pad).T

    br = 640
    grid = (n_pad // br,)
    dinv, xs = pl.pallas_call(
        _norm_body,
        grid=grid,
        in_specs=[
            pl.BlockSpec((br, NC), lambda i: (i, 0)),
            pl.BlockSpec((br, d_in), lambda i: (i, 0)),
        ],
        out_specs=[
            pl.BlockSpec((br, 1), lambda i: (i, 0)),
            pl.BlockSpec((br, d_in), lambda i: (i, 0)),
        ],
        out_shape=[
            jax.ShapeDtypeStruct((n_pad, 1), jnp.float32),
            jax.ShapeDtypeStruct((n_pad, d_in), jnp.float32),
        ],
    )(degt, x_pad)

    aggp = _build_agg_kernel(n_pad, nb, nb_tot, d_in)(src_full, dst_full, xs)
    aggp = aggp.reshape(NC, n_pad, d_in)

    out_pad = pl.pallas_call(
        _out_body,
        grid=grid,
        in_specs=[
            pl.BlockSpec((NC, br, d_in), lambda i: (0, i, 0)),
            pl.BlockSpec((br, d_in), lambda i: (i, 0)),
            pl.BlockSpec((br, 1), lambda i: (i, 0)),
            pl.BlockSpec((d_in, d_hid), lambda i: (0, 0)),
            pl.BlockSpec((1, d_hid), lambda i: (0, 0)),
        ],
        out_specs=pl.BlockSpec((br, d_hid), lambda i: (i, 0)),
        out_shape=jax.ShapeDtypeStruct((n_pad, d_hid), jnp.float32),
    )(aggp, xs, dinv, W1, b1.reshape(1, d_hid))

    return out_pad[:n]


# speedup vs baseline: 44.7308x; 44.7308x over previous
import jax
import jax.numpy as jnp
from jax import lax
from jax.experimental import pallas as pl
from jax.experimental.pallas import tpu as pltpu
from jax.experimental.pallas import tpu_sc as plsc

NC = 2
NS = 16
NW = NC * NS
LANE = 16
BATCH = 128
BL = BATCH // LANE


def _ceil_to(a, m):
    return -(-a // m) * m


def _build_deg_kernel(n_pad, nb, nb_tot, shift):
    rows_t = n_pad // NS
    mask = (1 << shift) - 1
    mesh = plsc.VectorSubcoreMesh(core_axis_name="c", subcore_axis_name="s")

    def body(pk_hbm, out_hbm, pk_v, dst_v, ones_v, stage_v, deg_sh):
        c = lax.axis_index("c")
        s = lax.axis_index("s")
        wid = c * NS + s

        def fill_ones(i, carry):
            ones_v[pl.ds(i * LANE, LANE)] = jnp.ones((LANE,), jnp.float32)
            return carry

        lax.fori_loop(0, BATCH // LANE, fill_ones, 0)

        def fill_zero(i, carry):
            stage_v[pl.ds(i * LANE, LANE)] = jnp.zeros((LANE,), jnp.float32)
            return carry

        lax.fori_loop(0, rows_t // LANE, fill_zero, 0)

        pltpu.sync_copy(stage_v, deg_sh.at[pl.ds(s * rows_t, rows_t)])
        pltpu.sync_copy(pk_hbm.at[pl.ds(wid * nb_tot, nb)], pk_v)

        def unpack(i, carry):
            r = i // BL
            col = (i % BL) * LANE
            v = pk_v[r, pl.ds(col, LANE)]
            dst_v[r, pl.ds(col, LANE)] = lax.bitwise_and(v, mask)
            return carry

        lax.fori_loop(0, nb * BL, unpack, 0)
        plsc.subcore_barrier()

        def scat(b, carry):
            pltpu.sync_copy(ones_v, deg_sh.at[dst_v.at[b]], add=True)
            return carry

        lax.fori_loop(0, nb, scat, 0)
        plsc.subcore_barrier()

        pltpu.sync_copy(deg_sh.at[pl.ds(s * rows_t, rows_t)], stage_v)
        pltpu.sync_copy(stage_v, out_hbm.at[pl.ds(wid * rows_t, rows_t)])

    return pl.kernel(
        body,
        out_type=jax.ShapeDtypeStruct((NC * n_pad,), jnp.float32),
        mesh=mesh,
        scratch_types=[
            pltpu.VMEM((nb, BATCH), jnp.int32),
            pltpu.VMEM((nb, BATCH), jnp.int32),
            pltpu.VMEM((BATCH,), jnp.float32),
            pltpu.VMEM((rows_t,), jnp.float32),
            pltpu.VMEM_SHARED((n_pad,), jnp.float32),
        ],
    )


def _build_agg_kernel(n_pad, nb, nb_tot, d, shift):
    rows_t = n_pad // NS
    n_cp = rows_t // BATCH
    mask = (1 << shift) - 1
    mesh = plsc.VectorSubcoreMesh(core_axis_name="c", subcore_axis_name="s")

    def body(pk_hbm, xs_hbm, out_hbm,
             pk_v, si_v, di_v, buf0, buf1, agg_sh, sem0, sem1):
        c = lax.axis_index("c")
        s = lax.axis_index("s")
        wid = c * NS + s
        dl = d // LANE

        def z(i, carry):
            buf0[i // dl, pl.ds((i % dl) * LANE, LANE)] = (
                jnp.zeros((LANE,), jnp.float32))
            return carry

        lax.fori_loop(0, BATCH * dl, z, 0)

        def zc(i, carry):
            pltpu.sync_copy(buf0, agg_sh.at[pl.ds(s * rows_t + i * BATCH, BATCH)])
            return carry

        lax.fori_loop(0, n_cp, zc, 0)

        pltpu.sync_copy(pk_hbm.at[pl.ds(wid * nb_tot, nb_tot)], pk_v)
        plsc.subcore_barrier()

        def upk_src(b, carry):
            r = b % 4

            def col(j, c2):
                v = pk_v[b, pl.ds(j * LANE, LANE)]
                si_v[r, pl.ds(j * LANE, LANE)] = (
                    lax.shift_right_logical(v, shift))
                return c2

            return lax.fori_loop(0, BL, col, carry)

        def upk_dst(b, carry):
            r = b % 4

            def col(j, c2):
                v = pk_v[b, pl.ds(j * LANE, LANE)]
                di_v[r, pl.ds(j * LANE, LANE)] = lax.bitwise_and(v, mask)
                return c2

            return lax.fori_loop(0, BL, col, carry)

        upk_src(0, 0)
        upk_src(1, 0)
        pltpu.async_copy(xs_hbm.at[si_v.at[0]], buf0, sem0)
        pltpu.async_copy(xs_hbm.at[si_v.at[1]], buf1, sem1)

        def step(g, carry):
            b0 = 2 * g
            b1 = b0 + 1
            pltpu.make_async_copy(xs_hbm.at[si_v.at[b0 % 4]], buf0,
                                  sem0).wait()
            upk_src(b0 + 2, 0)
            upk_dst(b0, 0)
            pltpu.sync_copy(buf0, agg_sh.at[di_v.at[b0 % 4]], add=True)
            pltpu.async_copy(xs_hbm.at[si_v.at[(b0 + 2) % 4]], buf0, sem0)
            pltpu.make_async_copy(xs_hbm.at[si_v.at[b1 % 4]], buf1,
                                  sem1).wait()
            upk_src(b1 + 2, 0)
            upk_dst(b1, 0)
            pltpu.sync_copy(buf1, agg_sh.at[di_v.at[b1 % 4]], add=True)
            pltpu.async_copy(xs_hbm.at[si_v.at[(b1 + 2) % 4]], buf1, sem1)
            return carry

        lax.fori_loop(0, nb // 2, step, 0)
        pltpu.make_async_copy(xs_hbm.at[si_v.at[0]], buf0, sem0).wait()
        pltpu.make_async_copy(xs_hbm.at[si_v.at[1]], buf1, sem1).wait()
        plsc.subcore_barrier()

        def cp(i, carry):
            pltpu.sync_copy(agg_sh.at[pl.ds(s * rows_t + i * BATCH, BATCH)], buf0)
            pltpu.sync_copy(buf0, out_hbm.at[pl.ds(wid * rows_t + i * BATCH, BATCH)])
            return carry

        lax.fori_loop(0, n_cp, cp, 0)

    return pl.kernel(
        body,
        out_type=jax.ShapeDtypeStruct((NC * n_pad, d), jnp.float32),
        mesh=mesh,
        scratch_types=[
            pltpu.VMEM((nb_tot, BATCH), jnp.int32),
            pltpu.VMEM((4, BATCH), jnp.int32),
            pltpu.VMEM((4, BATCH), jnp.int32),
            pltpu.VMEM((BATCH, d), jnp.float32),
            pltpu.VMEM((BATCH, d), jnp.float32),
            pltpu.VMEM_SHARED((n_pad, d), jnp.float32),
            pltpu.SemaphoreType.DMA,
            pltpu.SemaphoreType.DMA,
        ],
    )


def _norm_body(degt_ref, x_ref, dinv_ref, xs_ref):
    deg = degt_ref[:, 0:1] + degt_ref[:, 1:2] + 1.0
    dinv = lax.rsqrt(deg)
    dinv_ref[...] = dinv
    xs_ref[...] = x_ref[...] * dinv


def _out_body(p_ref, xs_ref, dinv_ref, w_ref, b_ref, o_ref):
    m = p_ref[0] + p_ref[1] + xs_ref[...]
    y = jnp.dot(m, w_ref[...], preferred_element_type=jnp.float32)
    o_ref[...] = jnp.maximum(y * dinv_ref[...] + b_ref[...], 0.0)


def kernel(x, edge_index, W1, b1):
    n, d_in = x.shape
    d_hid = W1.shape[1]
    e = edge_index.shape[1]

    n_pad = _ceil_to(n + 1, NS * BATCH)
    pad_rows = n_pad - n
    shift = max(int(n_pad - 1).bit_length(), 1)
    nb = _ceil_to(-(-e // (NW * BATCH)), 8)
    e_pad = NW * nb * BATCH
    nb_tot = _ceil_to(nb + 2, 8)

    src = edge_index[0]
    dst = edge_index[1]
    pad_idx = n + (jnp.arange(e_pad - e, dtype=jnp.int32) % pad_rows)
    src_p = jnp.concatenate([src, pad_idx]).reshape(NW, nb, BATCH)
    dst_p = jnp.concatenate([dst, pad_idx]).reshape(NW, nb, BATCH)
    n_dummy = nb_tot - nb
    dummy = n + (jnp.arange(n_dummy * BATCH, dtype=jnp.int32) % pad_rows)
    dummy = jnp.broadcast_to(
        dummy.reshape(1, n_dummy, BATCH), (NW, n_dummy, BATCH))
    src_full = jnp.concatenate([src_p, dummy], axis=1)
    dst_full = jnp.concatenate([dst_p, dummy], axis=1)
    pk_full = (
        jnp.left_shift(src_full, shift) | dst_full
    ).reshape(NW * nb_tot, BATCH)
    x_pad = jnp.pad(x, ((0, pad_rows), (0, 0)))

    degp = _build_deg_kernel(n_pad, nb, nb_tot, shift)(pk_full)
    degt = degp.reshape(NC, n_pad).T

    br = 640
    grid = (n_pad // br,)
    dinv, xs = pl.pallas_call(
        _norm_body,
        grid=grid,
        in_specs=[
            pl.BlockSpec((br, NC), lambda i: (i, 0)),
            pl.BlockSpec((br, d_in), lambda i: (i, 0)),
        ],
        out_specs=[
            pl.BlockSpec((br, 1), lambda i: (i, 0)),
            pl.BlockSpec((br, d_in), lambda i: (i, 0)),
        ],
        out_shape=[
            jax.ShapeDtypeStruct((n_pad, 1), jnp.float32),
            jax.ShapeDtypeStruct((n_pad, d_in), jnp.float32),
        ],
    )(degt, x_pad)

    aggp = _build_agg_kernel(n_pad, nb, nb_tot, d_in, shift)(pk_full, xs)
    aggp = aggp.reshape(NC, n_pad, d_in)

    out_pad = pl.pallas_call(
        _out_body,
        grid=grid,
        in_specs=[
            pl.BlockSpec((NC, br, d_in), lambda i: (0, i, 0)),
            pl.BlockSpec((br, d_in), lambda i: (i, 0)),
            pl.BlockSpec((br, 1), lambda i: (i, 0)),
            pl.BlockSpec((d_in, d_hid), lambda i: (0, 0)),
            pl.BlockSpec((1, d_hid), lambda i: (0, 0)),
        ],
        out_specs=pl.BlockSpec((br, d_hid), lambda i: (i, 0)),
        out_shape=jax.ShapeDtypeStruct((n_pad, d_hid), jnp.float32),
    )(aggp, xs, dinv, W1, b1.reshape(1, d_hid))

    return out_pad[:n]
